# baseline (speedup 1.0000x reference)
import functools

import jax
import jax.numpy as jnp
from jax import lax
from jax.experimental import pallas as pl
from jax.experimental.pallas import tpu as pltpu
from jax.experimental.pallas import tpu_sc as plsc

B, S, HID, MAXN = 32, 512, 1024, 512
HALF = HID // 2
L = 16
NC, NS = 2, 16
NW = NC * NS
C = 32
NCHUNK = S // C
EPS = 1e-12


def _lane_sum(v):
    idx0 = jnp.arange(L, dtype=jnp.int32)
    dnums = lax.GatherDimensionNumbers(
        offset_dims=(), collapsed_slice_dims=(0,), start_index_map=(0,))
    for k in (8, 4, 2, 1):
        rot = lax.gather(v, ((idx0 + k) % L)[:, None], dnums, (1,),
                         mode=lax.GatherScatterMode.PROMISE_IN_BOUNDS)
        v = v + rot
    return v


def _rsqrt_vec(x):
    i = lax.bitcast_convert_type(x, jnp.int32)
    i = jnp.full((L,), 0x5F3759DF, jnp.int32) - (i >> 1)
    y = lax.bitcast_convert_type(i, jnp.float32)
    for _ in range(3):
        y = y * (1.5 - 0.5 * x * y * y)
    return y


_mesh = plsc.VectorSubcoreMesh(core_axis_name="c", subcore_axis_name="s")


@functools.partial(
    pl.kernel,
    out_type=jax.ShapeDtypeStruct((B * 2 * S, HALF), jnp.float32),
    mesh=_mesh,
    scratch_types=[
        pltpu.VMEM((2 * S,), jnp.int32),
        pltpu.VMEM((2 * C, HALF), jnp.float32),
        pltpu.VMEM((2 * C, HALF), jnp.float32),
        pltpu.VMEM((HID,), jnp.float32),
        pltpu.VMEM((HID,), jnp.float32),
        pltpu.VMEM((C, L), jnp.float32),
        pltpu.VMEM((C, L), jnp.float32),
        pltpu.SemaphoreType.DMA,
    ],
)
def _sc_kernel(comb_hbm, pos2_hbm, idx_hbm, gam_hbm, bet_hbm, out_hbm,
               idx_v, g_buf, p_buf, gam_v, bet_v, m_v, rs_v, sem):
    w = lax.axis_index("s") * NC + lax.axis_index("c")
    base = pl.multiple_of(w * (2 * S), 2 * S)
    pltpu.sync_copy(idx_hbm.at[pl.ds(base, 2 * S)], idx_v)
    pltpu.sync_copy(gam_hbm, gam_v)
    pltpu.sync_copy(bet_hbm, bet_v)

    def chunk_body(ci, carry):
        r0 = pl.multiple_of(ci * (2 * C), 2 * C)
        pltpu.sync_copy(pos2_hbm.at[pl.ds(r0, 2 * C)], p_buf)
        pltpu.async_copy(
            comb_hbm.at[idx_v.at[pl.ds(r0, 2 * C)]], g_buf, sem
        ).wait()

        def row_body(r, c1):
            acc_s = jnp.zeros((L,), jnp.float32)
            acc_q = jnp.zeros((L,), jnp.float32)
            for j in range(HID // L):
                row = 2 * r + (j // (HALF // L))
                col = (j % (HALF // L)) * L
                v = g_buf[row, pl.ds(col, L)] + p_buf[row, pl.ds(col, L)]
                g_buf[row, pl.ds(col, L)] = v
                acc_s = acc_s + v
                acc_q = acc_q + v * v
            s1 = _lane_sum(acc_s)
            s2 = _lane_sum(acc_q)
            mean = s1 * (1.0 / HID)
            var = s2 * (1.0 / HID) - mean * mean
            m_v[r, :] = mean
            rs_v[r, :] = _rsqrt_vec(var + EPS)
            return c1

        lax.fori_loop(0, C, row_body, 0)

        jper = 16
        for jb in range(HID // L // jper):
            gs = tuple(gam_v[pl.ds((jb * jper + t) * L, L)]
                       for t in range(jper))
            bs = tuple(bet_v[pl.ds((jb * jper + t) * L, L)]
                       for t in range(jper))

            def row2(r, carry, jb=jb):
                cgs, cbs = carry
                m = m_v[r, :]
                rs = rs_v[r, :]
                for t in range(jper):
                    j = jb * jper + t
                    h = j // (HALF // L)
                    col = (j % (HALF // L)) * L
                    e = g_buf[2 * r + h, pl.ds(col, L)]
                    g_buf[2 * r + h, pl.ds(col, L)] = (
                        (e - m) * rs * cgs[t] + cbs[t])
                return carry

            lax.fori_loop(0, C, row2, (gs, bs))

        pltpu.sync_copy(g_buf, out_hbm.at[pl.ds(base + r0, 2 * C)])
        return carry

    lax.fori_loop(0, NCHUNK, chunk_body, 0)


def kernel(top_vecs, tok_struct_vec, sent_struct_vec, pos_table, a_table,
           b_table, ln_gamma, ln_beta):
    del top_vecs, tok_struct_vec
    pa = sent_struct_vec[:, :, 0].astype(jnp.int32)
    sb = sent_struct_vec[:, :, 1].astype(jnp.int32) + MAXN
    idx = jnp.stack([pa, sb], axis=-1).reshape(B * 2 * S)
    comb = jnp.concatenate([a_table, b_table], axis=0)
    pos2 = pos_table.reshape(2 * S, HALF)
    out = _sc_kernel(comb, pos2, idx, ln_gamma, ln_beta)
    return out.reshape(B, S, HID)


# speedup vs baseline: 1.3378x; 1.3378x over previous
import functools

import jax
import jax.numpy as jnp
from jax import lax
from jax.experimental import pallas as pl
from jax.experimental.pallas import tpu as pltpu
from jax.experimental.pallas import tpu_sc as plsc

B, S, HID, MAXN = 32, 512, 1024, 512
HALF = HID // 2
L = 16
NC, NS = 2, 16
NW = NC * NS
C = 16
NCH = S // C
JV = HID // L
EPS = 1e-12


def _lane_sum(v):
    idx0 = jnp.arange(L, dtype=jnp.int32)
    dnums = lax.GatherDimensionNumbers(
        offset_dims=(), collapsed_slice_dims=(0,), start_index_map=(0,))
    for k in (8, 4, 2, 1):
        rot = lax.gather(v, ((idx0 + k) % L)[:, None], dnums, (1,),
                         mode=lax.GatherScatterMode.PROMISE_IN_BOUNDS)
        v = v + rot
    return v


def _rsqrt_vec(x):
    i = lax.bitcast_convert_type(x, jnp.int32)
    i = jnp.full((L,), 0x5F3759DF, jnp.int32) - (i >> 1)
    y = lax.bitcast_convert_type(i, jnp.float32)
    for _ in range(3):
        y = y * (1.5 - 0.5 * x * y * y)
    return y


_mesh = plsc.VectorSubcoreMesh(core_axis_name="c", subcore_axis_name="s")


@functools.partial(
    pl.kernel,
    out_type=jax.ShapeDtypeStruct((B * 2 * S, HALF), jnp.float32),
    mesh=_mesh,
    scratch_types=[
        pltpu.VMEM((2 * S,), jnp.int32),
        pltpu.VMEM((2 * C, HALF), jnp.float32),
        pltpu.VMEM((2 * C, HALF), jnp.float32),
        pltpu.VMEM((2 * C, HALF), jnp.float32),
        pltpu.VMEM((2 * C, HALF), jnp.float32),
        pltpu.VMEM((2 * C, HALF), jnp.float32),
        pltpu.VMEM((2 * C, HALF), jnp.float32),
        pltpu.VMEM((HID,), jnp.float32),
        pltpu.VMEM((HID,), jnp.float32),
        pltpu.VMEM((C, L), jnp.float32),
        pltpu.VMEM((C, L), jnp.float32),
        pltpu.SemaphoreType.DMA,
        pltpu.SemaphoreType.DMA,
        pltpu.SemaphoreType.DMA,
        pltpu.SemaphoreType.DMA,
        pltpu.SemaphoreType.DMA,
        pltpu.SemaphoreType.DMA,
    ],
)
def _sc_kernel(comb_hbm, pos2_hbm, idx_hbm, gam_hbm, bet_hbm, out_hbm,
               idx_v, g0, g1, p0, p1, o0, o1, gam_v, bet_v, m_v, rs_v,
               gsem0, gsem1, psem0, psem1, osem0, osem1):
    w = lax.axis_index("s") * NC + lax.axis_index("c")
    base = pl.multiple_of(w * (2 * S), 2 * S)
    pltpu.sync_copy(idx_hbm.at[pl.ds(base, 2 * S)], idx_v)
    pltpu.sync_copy(gam_hbm, gam_v)
    pltpu.sync_copy(bet_hbm, bet_v)

    def issue_in(ci, g_buf, p_buf, gsem, psem):
        r0 = pl.multiple_of(ci * (2 * C), 2 * C)
        pltpu.async_copy(pos2_hbm.at[pl.ds(r0, 2 * C)], p_buf, psem)
        pltpu.async_copy(comb_hbm.at[idx_v.at[pl.ds(r0, 2 * C)]], g_buf, gsem)

    def wait_in(g_buf, p_buf, gsem, psem):
        pltpu.make_async_copy(pos2_hbm.at[pl.ds(0, 2 * C)], p_buf, psem).wait()
        pltpu.make_async_copy(pos2_hbm.at[pl.ds(0, 2 * C)], g_buf, gsem).wait()

    def issue_out(ci, o_buf, osem):
        r0 = pl.multiple_of(ci * (2 * C), 2 * C)
        pltpu.async_copy(o_buf, out_hbm.at[pl.ds(base + r0, 2 * C)], osem)

    def wait_out(o_buf, osem):
        pltpu.make_async_copy(o_buf, out_hbm.at[pl.ds(0, 2 * C)], osem).wait()

    def pass1(g_buf, p_buf):
        def row_body(r, c1):
            a_s = [jnp.zeros((L,), jnp.float32) for _ in range(4)]
            a_q = [jnp.zeros((L,), jnp.float32) for _ in range(4)]
            for j in range(JV):
                row = 2 * r + (j // (HALF // L))
                col = (j % (HALF // L)) * L
                v = g_buf[row, pl.ds(col, L)] + p_buf[row, pl.ds(col, L)]
                g_buf[row, pl.ds(col, L)] = v
                k = j % 4
                a_s[k] = a_s[k] + v
                a_q[k] = a_q[k] + v * v
            s1 = _lane_sum((a_s[0] + a_s[1]) + (a_s[2] + a_s[3]))
            s2 = _lane_sum((a_q[0] + a_q[1]) + (a_q[2] + a_q[3]))
            mean = s1 * (1.0 / HID)
            var = s2 * (1.0 / HID) - mean * mean
            m_v[r, :] = mean
            rs_v[r, :] = _rsqrt_vec(var + EPS)
            return c1

        lax.fori_loop(0, C, row_body, 0)

    def pass2(g_buf, o_buf):
        jper = 16
        for jb in range(JV // jper):
            gs = tuple(gam_v[pl.ds((jb * jper + t) * L, L)]
                       for t in range(jper))
            bs = tuple(bet_v[pl.ds((jb * jper + t) * L, L)]
                       for t in range(jper))

            def row2(r, carry, jb=jb):
                cgs, cbs = carry
                m = m_v[r, :]
                rs = rs_v[r, :]
                for t in range(jper):
                    j = jb * jper + t
                    row = 2 * r + (j // (HALF // L))
                    col = (j % (HALF // L)) * L
                    e = g_buf[row, pl.ds(col, L)]
                    o_buf[row, pl.ds(col, L)] = (e - m) * rs * cgs[t] + cbs[t]
                return carry

            lax.fori_loop(0, C, row2, (gs, bs))

    issue_in(0, g0, p0, gsem0, psem0)

    def body(t, carry):
        i0 = 2 * t
        issue_in(i0 + 1, g1, p1, gsem1, psem1)
        wait_in(g0, p0, gsem0, psem0)
        pass1(g0, p0)
        pl.when(t >= 1)(lambda: wait_out(o0, osem0))
        pass2(g0, o0)
        issue_out(i0, o0, osem0)
        pl.when(t < NCH // 2 - 1)(
            lambda: issue_in(i0 + 2, g0, p0, gsem0, psem0))
        wait_in(g1, p1, gsem1, psem1)
        pass1(g1, p1)
        pl.when(t >= 1)(lambda: wait_out(o1, osem1))
        pass2(g1, o1)
        issue_out(i0 + 1, o1, osem1)
        return carry

    lax.fori_loop(0, NCH // 2, body, 0)
    wait_out(o0, osem0)
    wait_out(o1, osem1)


def kernel(top_vecs, tok_struct_vec, sent_struct_vec, pos_table, a_table,
           b_table, ln_gamma, ln_beta):
    del top_vecs, tok_struct_vec
    pa = sent_struct_vec[:, :, 0].astype(jnp.int32)
    sb = sent_struct_vec[:, :, 1].astype(jnp.int32) + MAXN
    idx = jnp.stack([pa, sb], axis=-1).reshape(B * 2 * S)
    comb = jnp.concatenate([a_table, b_table], axis=0)
    pos2 = pos_table.reshape(2 * S, HALF)
    out = _sc_kernel(comb, pos2, idx, ln_gamma, ln_beta)
    return out.reshape(B, S, HID)
